# baseline (speedup 1.0000x reference)
import functools

import jax
import jax.numpy as jnp
from jax.experimental import pallas as pl
from jax.experimental.pallas import tpu as pltpu


def _se_step(x_ref, w1t_ref, w2t_ref, o_ref, *, inv_hw):
    xb = x_ref[...]
    pooled = jnp.sum(xb, axis=-1) * inv_hw
    h = jnp.dot(pooled, w1t_ref[...], preferred_element_type=jnp.float32)
    h = jnp.maximum(h, 0.0)
    logits = jnp.dot(h, w2t_ref[...], preferred_element_type=jnp.float32)
    gate = jax.nn.sigmoid(logits)
    o_ref[...] = xb * gate[:, :, None]


def kernel(x, w1, w2):
    B, C, H, W = x.shape
    HW = H * W
    c_r = w1.shape[0]

    x3 = x.reshape(B, C, HW)
    w1t = jnp.transpose(w1)
    w2t = jnp.transpose(w2)

    TB = 8
    while B % TB:
        TB //= 2
    grid = (B // TB,)

    out = pl.pallas_call(
        functools.partial(_se_step, inv_hw=1.0 / float(HW)),
        out_shape=jax.ShapeDtypeStruct((B, C, HW), x.dtype),
        grid=grid,
        in_specs=[
            pl.BlockSpec((TB, C, HW), lambda b: (b, 0, 0)),
            pl.BlockSpec((C, c_r), lambda b: (0, 0)),
            pl.BlockSpec((c_r, C), lambda b: (0, 0)),
        ],
        out_specs=pl.BlockSpec((TB, C, HW), lambda b: (b, 0, 0)),
        compiler_params=pltpu.CompilerParams(
            dimension_semantics=("parallel",),
            vmem_limit_bytes=64 << 20,
        ),
    )(x3, w1t, w2t)
    return out.reshape(B, C, H, W)
